# baseline (speedup 1.0000x reference)
import functools

import jax
import jax.numpy as jnp
from jax.experimental import pallas as pl

K = 16
RB = 128


def _bitonic_merge(vals, desc):
    n = len(vals)
    if n == 1:
        return vals
    half = n // 2
    out = list(vals)
    for i in range(half):
        hi = jnp.maximum(vals[i], vals[i + half])
        lo = jnp.minimum(vals[i], vals[i + half])
        if desc:
            out[i], out[i + half] = hi, lo
        else:
            out[i], out[i + half] = lo, hi
    return _bitonic_merge(out[:half], desc) + _bitonic_merge(out[half:], desc)


def _oe_merge(a, b, desc):
    if len(a) == 1 and len(b) == 1:
        hi = jnp.maximum(a[0], b[0])
        lo = jnp.minimum(a[0], b[0])
        return [hi, lo] if desc else [lo, hi]
    even = _oe_merge(a[0::2], b[0::2], desc)
    odd = _oe_merge(a[1::2], b[1::2], desc)
    out = [even[0]]
    for i in range(len(odd) - 1):
        hi = jnp.maximum(odd[i], even[i + 1])
        lo = jnp.minimum(odd[i], even[i + 1])
        if desc:
            out.extend([hi, lo])
        else:
            out.extend([lo, hi])
    out.append(odd[-1])
    return out


def _oe_sort(vals, desc):
    n = len(vals)
    if n == 1:
        return vals
    half = n // 2
    return _oe_merge(_oe_sort(vals[:half], desc), _oe_sort(vals[half:], desc), desc)


def _topk_kernel(x_ref, o_ref):
    s = x_ref.shape[1]
    cb = x_ref.shape[2]
    n_chunks = s // RB

    def body(i, acc):
        chunk = x_ref[0, pl.ds(i * RB, RB), :]
        c3 = chunk.reshape(K, 8, cb)
        vals = [c3[j] for j in range(K)]
        vals = _oe_sort(vals, True)
        merged = [jnp.maximum(acc[j], vals[K - 1 - j]) for j in range(K)]
        merged = _bitonic_merge(merged, True)
        return tuple(merged)

    acc0 = tuple(jnp.full((8, cb), -jnp.inf, jnp.float32) for _ in range(K))
    acc = jax.lax.fori_loop(0, n_chunks, body, acc0, unroll=2)

    lists = list(acc)
    w = 8
    while w > 1:
        half = w // 2
        a = [v[:half] for v in lists]
        b = [v[half:] for v in lists]
        merged = [jnp.maximum(a[j], b[K - 1 - j]) for j in range(K)]
        lists = _bitonic_merge(merged, True)
        w = half
    out = jnp.concatenate(lists, axis=0)
    o_ref[0] = out


def kernel(inputs):
    b, s, c = inputs.shape
    cb = min(c, 128)
    grid = (b, c // cb)
    out = pl.pallas_call(
        _topk_kernel,
        grid=grid,
        in_specs=[
            pl.BlockSpec((1, s, cb), lambda i, j: (i, 0, j)),
        ],
        out_specs=pl.BlockSpec((1, K, cb), lambda i, j: (i, 0, j)),
        out_shape=jax.ShapeDtypeStruct((b, K, c), jnp.float32),
    )(inputs)
    return jnp.transpose(out, (0, 2, 1)).reshape(b, c * K)


# speedup vs baseline: 1.0794x; 1.0794x over previous
import functools

import jax
import jax.numpy as jnp
from jax.experimental import pallas as pl

K = 16
RB = 128


def _bitonic_merge(vals, desc):
    n = len(vals)
    if n == 1:
        return vals
    half = n // 2
    out = list(vals)
    for i in range(half):
        hi = jnp.maximum(vals[i], vals[i + half])
        lo = jnp.minimum(vals[i], vals[i + half])
        if desc:
            out[i], out[i + half] = hi, lo
        else:
            out[i], out[i + half] = lo, hi
    return _bitonic_merge(out[:half], desc) + _bitonic_merge(out[half:], desc)


def _oe_merge(a, b, desc):
    if len(a) == 1 and len(b) == 1:
        hi = jnp.maximum(a[0], b[0])
        lo = jnp.minimum(a[0], b[0])
        return [hi, lo] if desc else [lo, hi]
    even = _oe_merge(a[0::2], b[0::2], desc)
    odd = _oe_merge(a[1::2], b[1::2], desc)
    out = [even[0]]
    for i in range(len(odd) - 1):
        hi = jnp.maximum(odd[i], even[i + 1])
        lo = jnp.minimum(odd[i], even[i + 1])
        if desc:
            out.extend([hi, lo])
        else:
            out.extend([lo, hi])
    out.append(odd[-1])
    return out


def _oe_sort(vals, desc):
    n = len(vals)
    if n == 1:
        return vals
    half = n // 2
    return _oe_merge(_oe_sort(vals[:half], desc), _oe_sort(vals[half:], desc), desc)


def _topk_kernel(x_ref, o_ref):
    s = x_ref.shape[1]
    cb = x_ref.shape[2]
    n_chunks = s // RB

    def body(i, acc):
        chunk = x_ref[0, pl.ds(i * RB, RB), :]
        c3 = chunk.reshape(K, 8, cb)
        vals = [c3[j] for j in range(K)]
        vals = _oe_sort(vals, True)
        merged = [jnp.maximum(acc[j], vals[K - 1 - j]) for j in range(K)]
        merged = _bitonic_merge(merged, True)
        return tuple(merged)

    acc0 = tuple(jnp.full((8, cb), -jnp.inf, jnp.float32) for _ in range(K))
    acc = jax.lax.fori_loop(0, n_chunks, body, acc0, unroll=4)

    lists = list(acc)
    w = 8
    while w > 1:
        half = w // 2
        a = [v[:half] for v in lists]
        b = [v[half:] for v in lists]
        merged = [jnp.maximum(a[j], b[K - 1 - j]) for j in range(K)]
        lists = _bitonic_merge(merged, True)
        w = half
    out = jnp.concatenate(lists, axis=0)
    o_ref[0] = out


def kernel(inputs):
    b, s, c = inputs.shape
    cb = min(c, 256)
    grid = (b, c // cb)
    out = pl.pallas_call(
        _topk_kernel,
        grid=grid,
        in_specs=[
            pl.BlockSpec((1, s, cb), lambda i, j: (i, 0, j)),
        ],
        out_specs=pl.BlockSpec((1, K, cb), lambda i, j: (i, 0, j)),
        out_shape=jax.ShapeDtypeStruct((b, K, c), jnp.float32),
    )(inputs)
    return jnp.transpose(out, (0, 2, 1)).reshape(b, c * K)
